# baseline (speedup 1.0000x reference)
import jax
import jax.numpy as jnp
from jax.experimental import pallas as pl
from jax.experimental.pallas import tpu as pltpu


def _outer4d_kernel(x_ref, o_ref):
    x = x_ref[0]
    o_ref[0] = (x[:, :, None] * x[:, None, :]).astype(o_ref.dtype)


def kernel(x):
    B, C, N = x.shape
    itemsize = x.dtype.itemsize

    target = 2 * 1024 * 1024
    tc = C
    if C * N * N * itemsize > target:
        cap = max(8, target // (N * N * itemsize))
        tc = (min(C, cap) // 8) * 8
        while tc > 8 and C % tc:
            tc -= 8

    return pl.pallas_call(
        _outer4d_kernel,
        out_shape=jax.ShapeDtypeStruct((B, C, N, N), x.dtype),
        grid=(B, C // tc),
        in_specs=[pl.BlockSpec((1, tc, N), lambda b, c: (b, c, 0))],
        out_specs=pl.BlockSpec((1, tc, N, N), lambda b, c: (b, c, 0, 0)),
        compiler_params=pltpu.CompilerParams(
            dimension_semantics=("parallel", "parallel"),
            vmem_limit_bytes=64 * 1024 * 1024,
        ),
        cost_estimate=pl.CostEstimate(
            flops=B * C * N * N,
            transcendentals=0,
            bytes_accessed=(B * C * N + B * C * N * N) * itemsize,
        ),
    )(x)


# speedup vs baseline: 1.6223x; 1.6223x over previous
import jax
import jax.numpy as jnp
from jax.experimental import pallas as pl
from jax.experimental.pallas import tpu as pltpu


def _outer_rows_kernel(x_ref, erep_ref, o_ref):
    x = x_ref[0]
    n = x.shape[1]
    rows = o_ref.shape[2]
    nn = rows * 128
    x2 = jnp.concatenate([x, x], axis=1)
    xj = pltpu.repeat(x2, nn // (2 * n), axis=1)
    xi = jnp.dot(x, erep_ref[...], preferred_element_type=jnp.float32)
    prod = (xi * xj).astype(o_ref.dtype)
    o_ref[0] = prod.reshape(x.shape[0], rows, 128)


def kernel(x):
    B, C, N = x.shape
    NN = N * N
    itemsize = x.dtype.itemsize

    target = 2 * 1024 * 1024
    tc = C
    if C * NN * itemsize > target:
        cap = max(8, target // (NN * itemsize))
        tc = (min(C, cap) // 8) * 8
        while tc > 8 and C % tc:
            tc -= 8

    e_rep = jnp.repeat(jnp.eye(N, dtype=x.dtype), N, axis=1)

    rows = NN // 128
    res = pl.pallas_call(
        _outer_rows_kernel,
        out_shape=jax.ShapeDtypeStruct((B, C, rows, 128), x.dtype),
        grid=(B, C // tc),
        in_specs=[
            pl.BlockSpec((1, tc, N), lambda b, c: (b, c, 0)),
            pl.BlockSpec((N, NN), lambda b, c: (0, 0)),
        ],
        out_specs=pl.BlockSpec((1, tc, rows, 128), lambda b, c: (b, c, 0, 0)),
        compiler_params=pltpu.CompilerParams(
            dimension_semantics=("parallel", "parallel"),
            vmem_limit_bytes=64 * 1024 * 1024,
        ),
        cost_estimate=pl.CostEstimate(
            flops=B * C * NN + 2 * B * C * N * NN,
            transcendentals=0,
            bytes_accessed=(B * C * N + N * NN + B * C * NN) * itemsize,
        ),
    )(x, e_rep)

    return res.reshape(B, C, N, N)


# speedup vs baseline: 5.3765x; 3.3141x over previous
import jax
import jax.numpy as jnp
from jax.experimental import pallas as pl
from jax.experimental.pallas import tpu as pltpu


def _outer_phys_kernel(x_ref, o_ref):
    ti = o_ref.shape[1]
    i0 = pl.multiple_of(pl.program_id(1) * ti, ti)
    xall = x_ref[0]
    xi = x_ref[0, pl.ds(i0, ti)]
    o_ref[0] = (xi[:, None, :] * xall[None, :, :]).astype(o_ref.dtype)


def kernel(x):
    B, C, N = x.shape
    itemsize = x.dtype.itemsize

    xp = jnp.transpose(x, (0, 2, 1))

    target = 2 * 1024 * 1024
    ti = N
    if N * N * C * itemsize > target:
        cap = max(8, target // (N * C * itemsize))
        ti = (min(N, cap) // 8) * 8
        while ti > 8 and N % ti:
            ti -= 8

    res = pl.pallas_call(
        _outer_phys_kernel,
        out_shape=jax.ShapeDtypeStruct((B, N, N, C), x.dtype),
        grid=(B, N // ti),
        in_specs=[pl.BlockSpec((1, N, C), lambda b, i: (b, 0, 0))],
        out_specs=pl.BlockSpec((1, ti, N, C), lambda b, i: (b, i, 0, 0)),
        compiler_params=pltpu.CompilerParams(
            dimension_semantics=("parallel", "parallel"),
            vmem_limit_bytes=64 * 1024 * 1024,
        ),
        cost_estimate=pl.CostEstimate(
            flops=B * C * N * N,
            transcendentals=0,
            bytes_accessed=(B * C * N + B * C * N * N) * itemsize,
        ),
    )(xp)

    return jnp.transpose(res, (0, 3, 1, 2))
